# baseline (speedup 1.0000x reference)
import functools

import jax
import jax.numpy as jnp
from jax import lax
from jax.experimental import pallas as pl
from jax.experimental.pallas import tpu as pltpu
from jax.experimental.pallas import tpu_sc as plsc

B = 16384
D = 128
F0 = 25
F1 = 10
NC = 2
NS = 16
NW = NC * NS
NPW = B // NW
LANES = 16
ND = D // LANES

GN = 4
F0P = 32
CH0 = GN * F0P
NCH0 = NPW // GN
SUB1 = 2
F1P = 16
CH1 = SUB1 * GN * F1P
NCH1 = NPW // (SUB1 * GN)
CHS = 128
NCHS = NPW // CHS


def _sc_body(nodes_h, n0_h, n1_h, emb_h, self_h, s0_h, s1_h,
             idxs_v, idx0_v, idx1_v, rows_v, out_v, sg0, sg1):
    wid = lax.axis_index("s") * NC + lax.axis_index("c")
    node_base = wid * NPW

    pltpu.sync_copy(nodes_h.at[pl.ds(wid * NPW, NPW)], idxs_v)
    pltpu.sync_copy(n0_h.at[pl.ds(wid * NCH0 * CH0, NCH0 * CH0)], idx0_v)
    pltpu.sync_copy(n1_h.at[pl.ds(wid * NCH1 * CH1, NCH1 * CH1)], idx1_v)

    sems = (sg0, sg1)

    def make_task(idx_v, ch):
        def gather(c, b):
            pltpu.async_copy(
                emb_h.at[idx_v.at[pl.ds(c * ch, ch)]],
                rows_v.at[b, pl.ds(0, ch)],
                sems[b],
            )

        def wait_gather(b):
            pltpu.make_async_copy(
                emb_h.at[idx_v.at[pl.ds(0, ch)]],
                rows_v.at[b, pl.ds(0, ch)],
                sems[b],
            ).wait()

        return gather, wait_gather

    def sum_group(b, rbase, fanout, nb):
        def seg(j, i, d):
            return rows_v[b, rbase + j * GN + i, pl.ds(d * LANES, LANES)]

        def jbody(j, accs):
            return tuple(
                accs[i * ND + d] + seg(j, i, d)
                for i in range(GN) for d in range(ND)
            )

        init = tuple(seg(0, i, d) for i in range(GN) for d in range(ND))
        accs = lax.fori_loop(1, fanout, jbody, init, unroll=2)
        for i in range(GN):
            for d in range(ND):
                out_v[nb + i, pl.ds(d * LANES, LANES)] = accs[i * ND + d]

    def run_task(idx_v, nch, ch, reduce_fn):
        gather, wait_gather = make_task(idx_v, ch)
        gather(0, 0)

        def pair(cp, _):
            c0 = cp * 2
            gather(c0 + 1, 1)
            wait_gather(0)
            reduce_fn(c0, 0)

            @pl.when(c0 + 2 < nch)
            def _():
                gather(c0 + 2, 0)

            wait_gather(1)
            reduce_fn(c0 + 1, 1)
            return 0

        lax.fori_loop(0, nch // 2, pair, 0)

    def self_reduce(c, b):
        pltpu.sync_copy(rows_v.at[b], self_h.at[pl.ds(node_base + c * CHS, CHS)])

    run_task(idxs_v, NCHS, CHS, self_reduce)

    def reduce0(c, b):
        sum_group(b, 0, F0, c * GN)

    run_task(idx0_v, NCH0, CH0, reduce0)
    pltpu.sync_copy(out_v, s0_h.at[pl.ds(node_base, NPW)])

    def reduce1(c, b):
        for s in range(SUB1):
            sum_group(b, s * GN * F1P, F1, c * SUB1 * GN + s * GN)

    run_task(idx1_v, NCH1, CH1, reduce1)
    pltpu.sync_copy(out_v, s1_h.at[pl.ds(node_base, NPW)])


@functools.cache
def _sc_gather():
    return pl.kernel(
        _sc_body,
        out_type=(
            jax.ShapeDtypeStruct((B, D), jnp.float32),
            jax.ShapeDtypeStruct((B, D), jnp.float32),
            jax.ShapeDtypeStruct((B, D), jnp.float32),
        ),
        mesh=plsc.VectorSubcoreMesh(
            core_axis_name="c", subcore_axis_name="s", num_cores=NC, num_subcores=NS
        ),
        scratch_types=(
            pltpu.VMEM((NPW,), jnp.int32),
            pltpu.VMEM((NCH0 * CH0,), jnp.int32),
            pltpu.VMEM((NCH1 * CH1,), jnp.int32),
            pltpu.VMEM((2, CHS, D), jnp.float32),
            pltpu.VMEM((NPW, D), jnp.float32),
            pltpu.SemaphoreType.DMA,
            pltpu.SemaphoreType.DMA,
        ),
    )


_BLK = 1024


def _mm_body(sv, s0r, s1r, ws0, wn0, b0r, ws1, wn1, b1r, o):
    dot = functools.partial(
        jnp.dot, preferred_element_type=jnp.float32, precision=lax.Precision.HIGHEST
    )
    h = dot(sv[...], ws0[...]) + dot(s0r[...], wn0[...] * (1.0 / F0)) + b0r[...]
    h = jnp.maximum(h, 0.0)
    o2 = dot(h, ws1[...]) + dot(s1r[...], wn1[...] * (1.0 / F1)) + b1r[...]
    o[...] = jnp.maximum(o2, 0.0)


def _tc_matmuls(self_v, s0, s1, Ws0, Wn0, b0, Ws1, Wn1, b1):
    big = pl.BlockSpec((_BLK, D), lambda i: (i, 0))
    w = pl.BlockSpec((D, D), lambda i: (0, 0))
    bias = pl.BlockSpec((1, D), lambda i: (0, 0))
    return pl.pallas_call(
        _mm_body,
        grid=(B // _BLK,),
        in_specs=[big, big, big, w, w, bias, w, w, bias],
        out_specs=big,
        out_shape=jax.ShapeDtypeStruct((B, D), jnp.float32),
    )(self_v, s0, s1, Ws0, Wn0, b0.reshape(1, D), Ws1, Wn1, b1.reshape(1, D))


def kernel(nodes, neigh_samples_0, neigh_samples_1, embedding,
           Ws0, Wn0, b0, Ws1, Wn1, b1):
    nodes1d = nodes.astype(jnp.int32)
    n0 = (neigh_samples_0.astype(jnp.int32)
          .reshape(NW, NCH0, GN, F0).transpose(0, 1, 3, 2))
    n0 = jnp.pad(n0, ((0, 0), (0, 0), (0, F0P - F0), (0, 0))).reshape(-1)
    n1 = (neigh_samples_1.astype(jnp.int32)
          .reshape(NW, NCH1, SUB1, GN, F1).transpose(0, 1, 2, 4, 3))
    n1 = jnp.pad(n1, ((0, 0), (0, 0), (0, 0), (0, F1P - F1), (0, 0))).reshape(-1)
    self_v, s0, s1 = _sc_gather()(nodes1d, n0, n1, embedding)
    return _tc_matmuls(self_v, s0, s1, Ws0, Wn0, b0, Ws1, Wn1, b1)


# speedup vs baseline: 18.5061x; 18.5061x over previous
import functools

import jax
import jax.numpy as jnp
from jax import lax
from jax.experimental import pallas as pl
from jax.experimental.pallas import tpu as pltpu
from jax.experimental.pallas import tpu_sc as plsc

B = 16384
D = 128
F0 = 25
F1 = 10
NC = 2
NS = 16
NW = NC * NS
NPW = B // NW
LANES = 16
ND = D // LANES

GN = 4
F0P = 32
CH0 = GN * F0P
NCH0 = NPW // GN
SUB1 = 2
F1P = 16
CH1 = SUB1 * GN * F1P
NCH1 = NPW // (SUB1 * GN)
CHS = 128
NCHS = NPW // CHS


def _sc_body(nodes_h, n0_h, n1_h, emb_h, self_h, s0_h, s1_h,
             idxs_v, idx0_v, idx1_v, rows_v, out_v, sg0, sg1):
    wid = lax.axis_index("s") * NC + lax.axis_index("c")
    node_base = wid * NPW

    pltpu.sync_copy(nodes_h.at[pl.ds(wid * NPW, NPW)], idxs_v)
    pltpu.sync_copy(n0_h.at[pl.ds(wid * NCH0 * CH0, NCH0 * CH0)], idx0_v)
    pltpu.sync_copy(n1_h.at[pl.ds(wid * NCH1 * CH1, NCH1 * CH1)], idx1_v)

    sems = (sg0, sg1)

    def make_task(idx_v, ch):
        def gather(c, b):
            pltpu.async_copy(
                emb_h.at[idx_v.at[pl.ds(c * ch, ch)]],
                rows_v.at[b, pl.ds(0, ch)],
                sems[b],
            )

        def wait_gather(b):
            pltpu.make_async_copy(
                emb_h.at[idx_v.at[pl.ds(0, ch)]],
                rows_v.at[b, pl.ds(0, ch)],
                sems[b],
            ).wait()

        return gather, wait_gather

    def sum_group(b, rbase, fanout, nb):
        def seg(j, i, d):
            return rows_v[b, rbase + j * GN + i, pl.ds(d * LANES, LANES)]

        def jbody(j, accs):
            return tuple(
                accs[i * ND + d] + seg(j, i, d)
                for i in range(GN) for d in range(ND)
            )

        init = tuple(seg(0, i, d) for i in range(GN) for d in range(ND))
        accs = lax.fori_loop(1, fanout, jbody, init, unroll=2)
        for i in range(GN):
            for d in range(ND):
                out_v[nb + i, pl.ds(d * LANES, LANES)] = accs[i * ND + d]

    def run_task(idx_v, nch, ch, reduce_fn):
        gather, wait_gather = make_task(idx_v, ch)
        gather(0, 0)

        def pair(cp, _):
            c0 = cp * 2
            gather(c0 + 1, 1)
            wait_gather(0)
            reduce_fn(c0, 0)

            @pl.when(c0 + 2 < nch)
            def _():
                gather(c0 + 2, 0)

            wait_gather(1)
            reduce_fn(c0 + 1, 1)
            return 0

        lax.fori_loop(0, nch // 2, pair, 0)

    def self_reduce(c, b):
        pltpu.sync_copy(rows_v.at[b], self_h.at[pl.ds(node_base + c * CHS, CHS)])

    run_task(idxs_v, NCHS, CHS, self_reduce)

    def reduce0(c, b):
        sum_group(b, 0, F0, c * GN)

    run_task(idx0_v, NCH0, CH0, reduce0)
    pltpu.sync_copy(out_v, s0_h.at[pl.ds(node_base, NPW)])

    def reduce1(c, b):
        for s in range(SUB1):
            sum_group(b, s * GN * F1P, F1, c * SUB1 * GN + s * GN)

    run_task(idx1_v, NCH1, CH1, reduce1)
    pltpu.sync_copy(out_v, s1_h.at[pl.ds(node_base, NPW)])


@functools.cache
def _sc_gather():
    return pl.kernel(
        _sc_body,
        out_type=(
            jax.ShapeDtypeStruct((B, D), jnp.float32),
            jax.ShapeDtypeStruct((B, D), jnp.float32),
            jax.ShapeDtypeStruct((B, D), jnp.float32),
        ),
        mesh=plsc.VectorSubcoreMesh(
            core_axis_name="c", subcore_axis_name="s", num_cores=NC, num_subcores=NS
        ),
        scratch_types=(
            pltpu.VMEM((NPW,), jnp.int32),
            pltpu.VMEM((NCH0 * CH0,), jnp.int32),
            pltpu.VMEM((NCH1 * CH1,), jnp.int32),
            pltpu.VMEM((2, CHS, D), jnp.float32),
            pltpu.VMEM((NPW, D), jnp.float32),
            pltpu.SemaphoreType.DMA,
            pltpu.SemaphoreType.DMA,
        ),
    )


_BLK = 1024


def _mm_body(sv, s0r, s1r, ws0, wn0, b0r, ws1, wn1, b1r, o):
    dot = functools.partial(
        jnp.dot, preferred_element_type=jnp.float32, precision=lax.Precision.HIGHEST
    )
    h = dot(sv[...], ws0[...]) + dot(s0r[...], wn0[...] * (1.0 / F0)) + b0r[...]
    h = jnp.maximum(h, 0.0)
    o2 = dot(h, ws1[...]) + dot(s1r[...], wn1[...] * (1.0 / F1)) + b1r[...]
    o[...] = jnp.maximum(o2, 0.0)


def _tc_matmuls(self_v, s0, s1, Ws0, Wn0, b0, Ws1, Wn1, b1):
    big = pl.BlockSpec((_BLK, D), lambda i: (i, 0))
    w = pl.BlockSpec((D, D), lambda i: (0, 0))
    bias = pl.BlockSpec((1, D), lambda i: (0, 0))
    return pl.pallas_call(
        _mm_body,
        grid=(B // _BLK,),
        in_specs=[big, big, big, w, w, bias, w, w, bias],
        out_specs=big,
        out_shape=jax.ShapeDtypeStruct((B, D), jnp.float32),
    )(self_v, s0, s1, Ws0, Wn0, b0.reshape(1, D), Ws1, Wn1, b1.reshape(1, D))


def kernel(nodes, neigh_samples_0, neigh_samples_1, embedding,
           Ws0, Wn0, b0, Ws1, Wn1, b1):
    nodes1d = nodes.astype(jnp.int32)
    n0 = (neigh_samples_0.astype(jnp.int32)
          .reshape(NW, NCH0, GN, F0).transpose(0, 1, 3, 2))
    n0 = jnp.concatenate([n0, n0[:, :, :F0P - F0, :]], axis=2).reshape(-1)
    n1 = (neigh_samples_1.astype(jnp.int32)
          .reshape(NW, NCH1, SUB1, GN, F1).transpose(0, 1, 2, 4, 3))
    n1 = jnp.concatenate([n1, n1[:, :, :, :F1P - F1, :]], axis=3).reshape(-1)
    self_v, s0, s1 = _sc_gather()(nodes1d, n0, n1, embedding)
    return _tc_matmuls(self_v, s0, s1, Ws0, Wn0, b0, Ws1, Wn1, b1)


# speedup vs baseline: 30.3170x; 1.6382x over previous
import functools

import jax
import jax.numpy as jnp
from jax import lax
from jax.experimental import pallas as pl
from jax.experimental.pallas import tpu as pltpu
from jax.experimental.pallas import tpu_sc as plsc

B = 16384
D = 128
F0 = 25
F1 = 10
NC = 2
NS = 16
NW = NC * NS
NPW = B // NW
LANES = 16
ND = D // LANES

GN = 4
F0P = 26
CH0 = GN * F0P
STRIDE = 128
NCH0 = NPW // GN
SUB1 = 2
CH1 = SUB1 * GN * F1
NCH1 = NPW // (SUB1 * GN)
CHS = 128
NCHS = NPW // CHS


def _sc_body(nodes_h, n0_h, n1_h, emb_h, self_h, s0_h, s1_h,
             idxs_v, idx0_v, idx1_v, rows_v, out_v, sg0, sg1):
    wid = lax.axis_index("s") * NC + lax.axis_index("c")
    node_base = wid * NPW

    pltpu.sync_copy(nodes_h.at[pl.ds(wid * NPW, NPW)], idxs_v)
    pltpu.sync_copy(n0_h.at[pl.ds(wid * NCH0 * STRIDE, NCH0 * STRIDE)], idx0_v)
    pltpu.sync_copy(n1_h.at[pl.ds(wid * NCH1 * STRIDE, NCH1 * STRIDE)], idx1_v)

    sems = (sg0, sg1)

    def make_task(idx_v, ch):
        def gather(c, b):
            pltpu.async_copy(
                emb_h.at[idx_v.at[pl.ds(c * STRIDE, ch)]],
                rows_v.at[b, pl.ds(0, ch)],
                sems[b],
            )

        def wait_gather(b):
            pltpu.make_async_copy(
                emb_h.at[idx_v.at[pl.ds(0, ch)]],
                rows_v.at[b, pl.ds(0, ch)],
                sems[b],
            ).wait()

        return gather, wait_gather

    def sum_group(b, rbase, fanout, nb):
        def seg(j, i, d):
            return rows_v[b, rbase + j * GN + i, pl.ds(d * LANES, LANES)]

        def jbody(j, accs):
            return tuple(
                accs[i * ND + d] + seg(j, i, d)
                for i in range(GN) for d in range(ND)
            )

        init = tuple(seg(0, i, d) for i in range(GN) for d in range(ND))
        accs = lax.fori_loop(1, fanout, jbody, init, unroll=2)
        for i in range(GN):
            for d in range(ND):
                out_v[nb + i, pl.ds(d * LANES, LANES)] = accs[i * ND + d]

    def run_task(idx_v, nch, ch, reduce_fn):
        gather, wait_gather = make_task(idx_v, ch)
        gather(0, 0)

        def pair(cp, _):
            c0 = cp * 2
            gather(c0 + 1, 1)
            wait_gather(0)
            reduce_fn(c0, 0)

            @pl.when(c0 + 2 < nch)
            def _():
                gather(c0 + 2, 0)

            wait_gather(1)
            reduce_fn(c0 + 1, 1)
            return 0

        lax.fori_loop(0, nch // 2, pair, 0)

    def self_reduce(c, b):
        pltpu.sync_copy(rows_v.at[b], self_h.at[pl.ds(node_base + c * CHS, CHS)])

    run_task(idxs_v, NCHS, CHS, self_reduce)

    def reduce0(c, b):
        sum_group(b, 0, F0, c * GN)

    run_task(idx0_v, NCH0, CH0, reduce0)
    pltpu.sync_copy(out_v, s0_h.at[pl.ds(node_base, NPW)])

    def reduce1(c, b):
        for s in range(SUB1):
            sum_group(b, s * GN * F1, F1, c * SUB1 * GN + s * GN)

    run_task(idx1_v, NCH1, CH1, reduce1)
    pltpu.sync_copy(out_v, s1_h.at[pl.ds(node_base, NPW)])


@functools.cache
def _sc_gather():
    return pl.kernel(
        _sc_body,
        out_type=(
            jax.ShapeDtypeStruct((B, D), jnp.float32),
            jax.ShapeDtypeStruct((B, D), jnp.float32),
            jax.ShapeDtypeStruct((B, D), jnp.float32),
        ),
        mesh=plsc.VectorSubcoreMesh(
            core_axis_name="c", subcore_axis_name="s", num_cores=NC, num_subcores=NS
        ),
        scratch_types=(
            pltpu.VMEM((NPW,), jnp.int32),
            pltpu.VMEM((NCH0 * STRIDE,), jnp.int32),
            pltpu.VMEM((NCH1 * STRIDE,), jnp.int32),
            pltpu.VMEM((2, CHS, D), jnp.float32),
            pltpu.VMEM((NPW, D), jnp.float32),
            pltpu.SemaphoreType.DMA,
            pltpu.SemaphoreType.DMA,
        ),
    )


_BLK = 1024


def _mm_body(sv, s0r, s1r, ws0, wn0, b0r, ws1, wn1, b1r, o):
    dot = functools.partial(
        jnp.dot, preferred_element_type=jnp.float32, precision=lax.Precision.HIGHEST
    )
    h = dot(sv[...], ws0[...]) + dot(s0r[...], wn0[...] * (1.0 / F0)) + b0r[...]
    h = jnp.maximum(h, 0.0)
    o2 = dot(h, ws1[...]) + dot(s1r[...], wn1[...] * (1.0 / F1)) + b1r[...]
    o[...] = jnp.maximum(o2, 0.0)


def _tc_matmuls(self_v, s0, s1, Ws0, Wn0, b0, Ws1, Wn1, b1):
    big = pl.BlockSpec((_BLK, D), lambda i: (i, 0))
    w = pl.BlockSpec((D, D), lambda i: (0, 0))
    bias = pl.BlockSpec((1, D), lambda i: (0, 0))
    return pl.pallas_call(
        _mm_body,
        grid=(B // _BLK,),
        in_specs=[big, big, big, w, w, bias, w, w, bias],
        out_specs=big,
        out_shape=jax.ShapeDtypeStruct((B, D), jnp.float32),
    )(self_v, s0, s1, Ws0, Wn0, b0.reshape(1, D), Ws1, Wn1, b1.reshape(1, D))


def kernel(nodes, neigh_samples_0, neigh_samples_1, embedding,
           Ws0, Wn0, b0, Ws1, Wn1, b1):
    nodes1d = nodes.astype(jnp.int32)
    n0 = (neigh_samples_0.astype(jnp.int32)
          .reshape(NW, NCH0, GN, F0).transpose(0, 1, 3, 2))
    n0 = jnp.concatenate([n0, n0[:, :, :F0P - F0, :]], axis=2)
    n0 = n0.reshape(NW, NCH0, CH0)
    n0 = jnp.pad(n0, ((0, 0), (0, 0), (0, STRIDE - CH0))).reshape(-1)
    n1 = (neigh_samples_1.astype(jnp.int32)
          .reshape(NW, NCH1, SUB1, GN, F1).transpose(0, 1, 2, 4, 3))
    n1 = n1.reshape(NW, NCH1, CH1)
    n1 = jnp.pad(n1, ((0, 0), (0, 0), (0, STRIDE - CH1))).reshape(-1)
    self_v, s0, s1 = _sc_gather()(nodes1d, n0, n1, embedding)
    return _tc_matmuls(self_v, s0, s1, Ws0, Wn0, b0, Ws1, Wn1, b1)


# speedup vs baseline: 33.2138x; 1.0955x over previous
import functools

import jax
import jax.numpy as jnp
from jax import lax
from jax.experimental import pallas as pl
from jax.experimental.pallas import tpu as pltpu
from jax.experimental.pallas import tpu_sc as plsc

B = 16384
D = 128
F0 = 25
F1 = 10
NC = 2
NS = 16
NW = NC * NS
NPW = B // NW
LANES = 16
ND = D // LANES

GN = 4
F0P = 26
CH0 = GN * F0P
STRIDE = 128
NCH0 = NPW // GN
SUB1 = 2
CH1 = SUB1 * GN * F1
NCH1 = NPW // (SUB1 * GN)
CHS = 128
NCHS = NPW // CHS


def _sc_body(nodes_h, n0_h, n1_h, emb_h, self_h, s0_h, s1_h,
             idxs_v, idx0_v, idx1_v, rows_v, out_v, sg0, sg1):
    wid = lax.axis_index("s") * NC + lax.axis_index("c")
    node_base = wid * NPW

    pltpu.sync_copy(nodes_h.at[pl.ds(wid * NPW, NPW)], idxs_v)
    pltpu.sync_copy(n0_h.at[pl.ds(wid * NCH0 * STRIDE, NCH0 * STRIDE)], idx0_v)
    pltpu.sync_copy(n1_h.at[pl.ds(wid * NCH1 * STRIDE, NCH1 * STRIDE)], idx1_v)

    sems = (sg0, sg1)

    def make_task(idx_v, ch):
        def gather(c, b):
            pltpu.async_copy(
                emb_h.at[idx_v.at[pl.ds(c * STRIDE, ch)]],
                rows_v.at[b, pl.ds(0, ch)],
                sems[b],
            )

        def wait_gather(b):
            pltpu.make_async_copy(
                emb_h.at[idx_v.at[pl.ds(0, ch)]],
                rows_v.at[b, pl.ds(0, ch)],
                sems[b],
            ).wait()

        return gather, wait_gather

    def sum_group(b, rbase, fanout, nb):
        def seg(j, i, d):
            return rows_v[b, rbase + j * GN + i, pl.ds(d * LANES, LANES)]

        def jbody(j, accs):
            return tuple(
                accs[i * ND + d] + seg(j, i, d)
                for i in range(GN) for d in range(ND)
            )

        init = tuple(seg(0, i, d) for i in range(GN) for d in range(ND))
        accs = lax.fori_loop(1, fanout, jbody, init, unroll=2)
        for i in range(GN):
            for d in range(ND):
                out_v[nb + i, pl.ds(d * LANES, LANES)] = accs[i * ND + d]

    def run_task(idx_v, nch, ch, reduce_fn):
        gather, wait_gather = make_task(idx_v, ch)
        gather(0, 0)

        def pair(cp, _):
            c0 = cp * 2
            gather(c0 + 1, 1)
            wait_gather(0)
            reduce_fn(c0, 0)

            @pl.when(c0 + 2 < nch)
            def _():
                gather(c0 + 2, 0)

            wait_gather(1)
            reduce_fn(c0 + 1, 1)
            return 0

        lax.fori_loop(0, nch // 2, pair, 0)

    def self_reduce(c, b):
        pltpu.sync_copy(rows_v.at[b], self_h.at[pl.ds(node_base + c * CHS, CHS)])

    run_task(idxs_v, NCHS, CHS, self_reduce)

    def reduce0(c, b):
        sum_group(b, 0, F0, c * GN)

    run_task(idx0_v, NCH0, CH0, reduce0)
    pltpu.sync_copy(out_v, s0_h.at[pl.ds(node_base, NPW)])

    def reduce1(c, b):
        for s in range(SUB1):
            sum_group(b, s * GN * F1, F1, c * SUB1 * GN + s * GN)

    run_task(idx1_v, NCH1, CH1, reduce1)
    pltpu.sync_copy(out_v, s1_h.at[pl.ds(node_base, NPW)])


@functools.cache
def _sc_gather():
    return pl.kernel(
        _sc_body,
        out_type=(
            jax.ShapeDtypeStruct((B, D), jnp.float32),
            jax.ShapeDtypeStruct((B, D), jnp.float32),
            jax.ShapeDtypeStruct((B, D), jnp.float32),
        ),
        mesh=plsc.VectorSubcoreMesh(
            core_axis_name="c", subcore_axis_name="s", num_cores=NC, num_subcores=NS
        ),
        scratch_types=(
            pltpu.VMEM((NPW,), jnp.int32),
            pltpu.VMEM((NCH0 * STRIDE,), jnp.int32),
            pltpu.VMEM((NCH1 * STRIDE,), jnp.int32),
            pltpu.VMEM((2, CHS, D), jnp.float32),
            pltpu.VMEM((NPW, D), jnp.float32),
            pltpu.SemaphoreType.DMA,
            pltpu.SemaphoreType.DMA,
        ),
    )


_BLK = 1024


def _mm_body(sv, s0r, s1r, ws0, wn0, b0r, ws1, wn1, b1r, o):
    dot = functools.partial(jnp.dot, preferred_element_type=jnp.float32)
    h = dot(sv[...], ws0[...]) + dot(s0r[...], wn0[...] * (1.0 / F0)) + b0r[...]
    h = jnp.maximum(h, 0.0)
    o2 = dot(h, ws1[...]) + dot(s1r[...], wn1[...] * (1.0 / F1)) + b1r[...]
    o[...] = jnp.maximum(o2, 0.0)


def _tc_matmuls(self_v, s0, s1, Ws0, Wn0, b0, Ws1, Wn1, b1):
    big = pl.BlockSpec((_BLK, D), lambda i: (i, 0))
    w = pl.BlockSpec((D, D), lambda i: (0, 0))
    bias = pl.BlockSpec((1, D), lambda i: (0, 0))
    return pl.pallas_call(
        _mm_body,
        grid=(B // _BLK,),
        in_specs=[big, big, big, w, w, bias, w, w, bias],
        out_specs=big,
        out_shape=jax.ShapeDtypeStruct((B, D), jnp.float32),
    )(self_v, s0, s1, Ws0, Wn0, b0.reshape(1, D), Ws1, Wn1, b1.reshape(1, D))


def kernel(nodes, neigh_samples_0, neigh_samples_1, embedding,
           Ws0, Wn0, b0, Ws1, Wn1, b1):
    nodes1d = nodes.astype(jnp.int32)
    n0 = (neigh_samples_0.astype(jnp.int32)
          .reshape(NW, NCH0, GN, F0).transpose(0, 1, 3, 2))
    n0 = jnp.concatenate([n0, n0[:, :, :F0P - F0, :]], axis=2)
    n0 = n0.reshape(NW, NCH0, CH0)
    n0 = jnp.pad(n0, ((0, 0), (0, 0), (0, STRIDE - CH0))).reshape(-1)
    n1 = (neigh_samples_1.astype(jnp.int32)
          .reshape(NW, NCH1, SUB1, GN, F1).transpose(0, 1, 2, 4, 3))
    n1 = n1.reshape(NW, NCH1, CH1)
    n1 = jnp.pad(n1, ((0, 0), (0, 0), (0, STRIDE - CH1))).reshape(-1)
    self_v, s0, s1 = _sc_gather()(nodes1d, n0, n1, embedding)
    return _tc_matmuls(self_v, s0, s1, Ws0, Wn0, b0, Ws1, Wn1, b1)


# speedup vs baseline: 38.5412x; 1.1604x over previous
import functools

import jax
import jax.numpy as jnp
from jax import lax
from jax.experimental import pallas as pl
from jax.experimental.pallas import tpu as pltpu
from jax.experimental.pallas import tpu_sc as plsc

B = 16384
D = 128
F0 = 25
F1 = 10
NC = 2
NS = 16
NW = NC * NS
NPW = B // NW
LANES = 16
ND = D // LANES

GN = 4
F0P = 26
CH0 = GN * F0P
STRIDE = 128
NCH0 = NPW // GN
SUB1 = 2
CH1 = SUB1 * GN * F1
NCH1 = NPW // (SUB1 * GN)
CHS = 128
NCHS = NPW // CHS


NB = 4
HPW = NPW // 2


def _sc_body(nodes_h, n0_h, n1_h, emb_h, self_h, s0_h, s1_h,
             idxs_v, idx0_v, idx1_v, rows_v, out_v, sg0, sg1, sg2, sg3):
    wid = lax.axis_index("s") * NC + lax.axis_index("c")
    node_base = wid * NPW

    pltpu.sync_copy(nodes_h.at[pl.ds(wid * NPW, NPW)], idxs_v)
    pltpu.sync_copy(n0_h.at[pl.ds(wid * NCH0 * STRIDE, NCH0 * STRIDE)], idx0_v)
    pltpu.sync_copy(n1_h.at[pl.ds(wid * NCH1 * STRIDE, NCH1 * STRIDE)], idx1_v)

    sems = (sg0, sg1, sg2, sg3)

    def make_task(idx_v, ch, cbase):
        def gather(c, b):
            pltpu.async_copy(
                emb_h.at[idx_v.at[pl.ds((cbase + c) * STRIDE, ch)]],
                rows_v.at[b, pl.ds(0, ch)],
                sems[b],
            )

        def wait_gather(b):
            pltpu.make_async_copy(
                emb_h.at[idx_v.at[pl.ds(0, ch)]],
                rows_v.at[b, pl.ds(0, ch)],
                sems[b],
            ).wait()

        return gather, wait_gather

    def sum_group(b, rbase, fanout, nb):
        def seg(j, i, d):
            return rows_v[b, rbase + j * GN + i, pl.ds(d * LANES, LANES)]

        def jbody(j, accs):
            return tuple(
                accs[i * ND + d] + seg(j, i, d)
                for i in range(GN) for d in range(ND)
            )

        init = tuple(seg(0, i, d) for i in range(GN) for d in range(ND))
        accs = lax.fori_loop(1, fanout, jbody, init, unroll=2)
        for i in range(GN):
            for d in range(ND):
                out_v[nb + i, pl.ds(d * LANES, LANES)] = accs[i * ND + d]

    def run_task(idx_v, nch, ch, reduce_fn, cbase=0):
        gather, wait_gather = make_task(idx_v, ch, cbase)
        for k in range(NB):
            gather(k, k)

        def quad(q, _):
            c0 = q * NB
            for k in range(NB):
                wait_gather(k)
                reduce_fn(c0 + k, k)

                @pl.when(c0 + k + NB < nch)
                def _():
                    gather(c0 + k + NB, k)

            return 0

        lax.fori_loop(0, nch // NB, quad, 0)

    def self_reduce(c, b):
        pltpu.sync_copy(rows_v.at[b, pl.ds(0, CHS)],
                        self_h.at[pl.ds(node_base + c * CHS, CHS)])

    run_task(idxs_v, NCHS, CHS, self_reduce)

    def reduce0(c, b):
        sum_group(b, 0, F0, c * GN)

    def reduce1(c, b):
        for s in range(SUB1):
            sum_group(b, s * GN * F1, F1, c * SUB1 * GN + s * GN)

    for p in range(2):
        run_task(idx0_v, NCH0 // 2, CH0, reduce0, cbase=p * (NCH0 // 2))
        pltpu.sync_copy(out_v, s0_h.at[pl.ds(node_base + p * HPW, HPW)])
    for p in range(2):
        run_task(idx1_v, NCH1 // 2, CH1, reduce1, cbase=p * (NCH1 // 2))
        pltpu.sync_copy(out_v, s1_h.at[pl.ds(node_base + p * HPW, HPW)])


@functools.cache
def _sc_gather():
    return pl.kernel(
        _sc_body,
        out_type=(
            jax.ShapeDtypeStruct((B, D), jnp.float32),
            jax.ShapeDtypeStruct((B, D), jnp.float32),
            jax.ShapeDtypeStruct((B, D), jnp.float32),
        ),
        mesh=plsc.VectorSubcoreMesh(
            core_axis_name="c", subcore_axis_name="s", num_cores=NC, num_subcores=NS
        ),
        scratch_types=(
            pltpu.VMEM((NPW,), jnp.int32),
            pltpu.VMEM((NCH0 * STRIDE,), jnp.int32),
            pltpu.VMEM((NCH1 * STRIDE,), jnp.int32),
            pltpu.VMEM((NB, CHS, D), jnp.float32),
            pltpu.VMEM((HPW, D), jnp.float32),
            pltpu.SemaphoreType.DMA,
            pltpu.SemaphoreType.DMA,
            pltpu.SemaphoreType.DMA,
            pltpu.SemaphoreType.DMA,
        ),
    )


_BLK = 1024


def _mm_body(sv, s0r, s1r, ws0, wn0, b0r, ws1, wn1, b1r, o):
    dot = functools.partial(jnp.dot, preferred_element_type=jnp.float32)
    h = dot(sv[...], ws0[...]) + dot(s0r[...], wn0[...] * (1.0 / F0)) + b0r[...]
    h = jnp.maximum(h, 0.0)
    o2 = dot(h, ws1[...]) + dot(s1r[...], wn1[...] * (1.0 / F1)) + b1r[...]
    o[...] = jnp.maximum(o2, 0.0)


def _tc_matmuls(self_v, s0, s1, Ws0, Wn0, b0, Ws1, Wn1, b1):
    big = pl.BlockSpec((_BLK, D), lambda i: (i, 0))
    w = pl.BlockSpec((D, D), lambda i: (0, 0))
    bias = pl.BlockSpec((1, D), lambda i: (0, 0))
    return pl.pallas_call(
        _mm_body,
        grid=(B // _BLK,),
        in_specs=[big, big, big, w, w, bias, w, w, bias],
        out_specs=big,
        out_shape=jax.ShapeDtypeStruct((B, D), jnp.float32),
    )(self_v, s0, s1, Ws0, Wn0, b0.reshape(1, D), Ws1, Wn1, b1.reshape(1, D))


def kernel(nodes, neigh_samples_0, neigh_samples_1, embedding,
           Ws0, Wn0, b0, Ws1, Wn1, b1):
    nodes1d = nodes.astype(jnp.int32)
    n0 = (neigh_samples_0.astype(jnp.int32)
          .reshape(NW, NCH0, GN, F0).transpose(0, 1, 3, 2))
    n0 = jnp.concatenate([n0, n0[:, :, :F0P - F0, :]], axis=2)
    n0 = n0.reshape(NW, NCH0, CH0)
    n0 = jnp.pad(n0, ((0, 0), (0, 0), (0, STRIDE - CH0))).reshape(-1)
    n1 = (neigh_samples_1.astype(jnp.int32)
          .reshape(NW, NCH1, SUB1, GN, F1).transpose(0, 1, 2, 4, 3))
    n1 = n1.reshape(NW, NCH1, CH1)
    n1 = jnp.pad(n1, ((0, 0), (0, 0), (0, STRIDE - CH1))).reshape(-1)
    self_v, s0, s1 = _sc_gather()(nodes1d, n0, n1, embedding)
    return _tc_matmuls(self_v, s0, s1, Ws0, Wn0, b0, Ws1, Wn1, b1)


# speedup vs baseline: 39.2360x; 1.0180x over previous
import functools

import jax
import jax.numpy as jnp
from jax import lax
from jax.experimental import pallas as pl
from jax.experimental.pallas import tpu as pltpu
from jax.experimental.pallas import tpu_sc as plsc

B = 16384
D = 128
F0 = 25
F1 = 10
NC = 2
NS = 16
NW = NC * NS
NPW = B // NW
LANES = 16
ND = D // LANES

GN = 4
F0P = 26
CH0 = GN * F0P
STRIDE = 128
NCH0 = NPW // GN
SUB1 = 2
CH1 = SUB1 * GN * F1
NCH1 = NPW // (SUB1 * GN)
CHS = 128
NCHS = NPW // CHS

NB = 4
HPW = NPW // 2


def _make_machinery(emb_h, rows_v, out_v, sems):
    def make_task(idx_v, ch, cbase):
        def gather(c, b):
            pltpu.async_copy(
                emb_h.at[idx_v.at[pl.ds((cbase + c) * STRIDE, ch)]],
                rows_v.at[b, pl.ds(0, ch)],
                sems[b],
            )

        def wait_gather(b):
            pltpu.make_async_copy(
                emb_h.at[idx_v.at[pl.ds(0, ch)]],
                rows_v.at[b, pl.ds(0, ch)],
                sems[b],
            ).wait()

        return gather, wait_gather

    def sum_group(b, rbase, fanout, nb):
        def seg(j, i, d):
            return rows_v[b, rbase + j * GN + i, pl.ds(d * LANES, LANES)]

        def jbody(j, accs):
            return tuple(
                accs[i * ND + d] + seg(j, i, d)
                for i in range(GN) for d in range(ND)
            )

        init = tuple(seg(0, i, d) for i in range(GN) for d in range(ND))
        accs = lax.fori_loop(1, fanout, jbody, init, unroll=2)
        for i in range(GN):
            for d in range(ND):
                out_v[nb + i, pl.ds(d * LANES, LANES)] = accs[i * ND + d]

    def run_task(idx_v, nch, ch, reduce_fn, cbase=0):
        gather, wait_gather = make_task(idx_v, ch, cbase)
        for k in range(NB):
            gather(k, k)

        def quad(q, _):
            c0 = q * NB
            for k in range(NB):
                wait_gather(k)
                reduce_fn(c0 + k, k)

                @pl.when(c0 + k + NB < nch)
                def _():
                    gather(c0 + k + NB, k)

            return 0

        lax.fori_loop(0, nch // NB, quad, 0)

    return sum_group, run_task


def _sc_body_a(nodes_h, n0_h, emb_h, self_h, s0_h,
               idxs_v, idx0_v, rows_v, out_v, sg0, sg1, sg2, sg3):
    wid = lax.axis_index("s") * NC + lax.axis_index("c")
    node_base = wid * NPW

    pltpu.sync_copy(nodes_h.at[pl.ds(wid * NPW, NPW)], idxs_v)
    pltpu.sync_copy(n0_h.at[pl.ds(wid * NCH0 * STRIDE, NCH0 * STRIDE)], idx0_v)

    sum_group, run_task = _make_machinery(emb_h, rows_v, out_v,
                                          (sg0, sg1, sg2, sg3))

    def self_reduce(c, b):
        pltpu.sync_copy(rows_v.at[b, pl.ds(0, CHS)],
                        self_h.at[pl.ds(node_base + c * CHS, CHS)])

    run_task(idxs_v, NCHS, CHS, self_reduce)

    def reduce0(c, b):
        sum_group(b, 0, F0, c * GN)

    for p in range(2):
        run_task(idx0_v, NCH0 // 2, CH0, reduce0, cbase=p * (NCH0 // 2))
        pltpu.sync_copy(out_v, s0_h.at[pl.ds(node_base + p * HPW, HPW)])


def _sc_body_b(n1_h, emb_h, s1_h, idx1_v, rows_v, out_v, sg0, sg1, sg2, sg3):
    wid = lax.axis_index("s") * NC + lax.axis_index("c")
    node_base = wid * NPW

    pltpu.sync_copy(n1_h.at[pl.ds(wid * NCH1 * STRIDE, NCH1 * STRIDE)], idx1_v)

    sum_group, run_task = _make_machinery(emb_h, rows_v, out_v,
                                          (sg0, sg1, sg2, sg3))

    def reduce1(c, b):
        for s in range(SUB1):
            sum_group(b, s * GN * F1, F1, c * SUB1 * GN + s * GN)

    for p in range(2):
        run_task(idx1_v, NCH1 // 2, CH1, reduce1, cbase=p * (NCH1 // 2))
        pltpu.sync_copy(out_v, s1_h.at[pl.ds(node_base + p * HPW, HPW)])


def _sc_mesh():
    return plsc.VectorSubcoreMesh(
        core_axis_name="c", subcore_axis_name="s", num_cores=NC, num_subcores=NS
    )


_SEMS = (pltpu.SemaphoreType.DMA,) * NB


@functools.cache
def _sc_gather_a():
    return pl.kernel(
        _sc_body_a,
        out_type=(
            jax.ShapeDtypeStruct((B, D), jnp.float32),
            jax.ShapeDtypeStruct((B, D), jnp.float32),
        ),
        mesh=_sc_mesh(),
        scratch_types=(
            pltpu.VMEM((NPW,), jnp.int32),
            pltpu.VMEM((NCH0 * STRIDE,), jnp.int32),
            pltpu.VMEM((NB, CHS, D), jnp.float32),
            pltpu.VMEM((HPW, D), jnp.float32),
        ) + _SEMS,
    )


@functools.cache
def _sc_gather_b():
    return pl.kernel(
        _sc_body_b,
        out_type=jax.ShapeDtypeStruct((B, D), jnp.float32),
        mesh=_sc_mesh(),
        scratch_types=(
            pltpu.VMEM((NCH1 * STRIDE,), jnp.int32),
            pltpu.VMEM((NB, CHS, D), jnp.float32),
            pltpu.VMEM((HPW, D), jnp.float32),
        ) + _SEMS,
    )


_BLK = 1024


def _layer_body(scale):
    def body(xr, sr, wx, wn, br, o):
        dot = functools.partial(jnp.dot, preferred_element_type=jnp.float32)
        o[...] = jnp.maximum(
            dot(xr[...], wx[...]) + dot(sr[...], wn[...] * scale) + br[...], 0.0
        )

    return body


def _tc_layer(x, s, wx, wn, b, scale):
    big = pl.BlockSpec((_BLK, D), lambda i: (i, 0))
    w = pl.BlockSpec((D, D), lambda i: (0, 0))
    bias = pl.BlockSpec((1, D), lambda i: (0, 0))
    return pl.pallas_call(
        _layer_body(scale),
        grid=(B // _BLK,),
        in_specs=[big, big, w, w, bias],
        out_specs=big,
        out_shape=jax.ShapeDtypeStruct((B, D), jnp.float32),
    )(x, s, wx, wn, b.reshape(1, D))


def kernel(nodes, neigh_samples_0, neigh_samples_1, embedding,
           Ws0, Wn0, b0, Ws1, Wn1, b1):
    nodes1d = nodes.astype(jnp.int32)
    n0 = (neigh_samples_0.astype(jnp.int32)
          .reshape(NW, NCH0, GN, F0).transpose(0, 1, 3, 2))
    n0 = jnp.concatenate([n0, n0[:, :, :F0P - F0, :]], axis=2)
    n0 = n0.reshape(NW, NCH0, CH0)
    n0 = jnp.pad(n0, ((0, 0), (0, 0), (0, STRIDE - CH0))).reshape(-1)
    n1 = (neigh_samples_1.astype(jnp.int32)
          .reshape(NW, NCH1, SUB1, GN, F1).transpose(0, 1, 2, 4, 3))
    n1 = n1.reshape(NW, NCH1, CH1)
    n1 = jnp.pad(n1, ((0, 0), (0, 0), (0, STRIDE - CH1))).reshape(-1)

    self_v, s0 = _sc_gather_a()(nodes1d, n0, embedding)
    s1 = _sc_gather_b()(n1, embedding)
    h = _tc_layer(self_v, s0, Ws0, Wn0, b0, 1.0 / F0)
    return _tc_layer(h, s1, Ws1, Wn1, b1, 1.0 / F1)


# speedup vs baseline: 41.1003x; 1.0475x over previous
import functools

import jax
import jax.numpy as jnp
from jax import lax
from jax.experimental import pallas as pl
from jax.experimental.pallas import tpu as pltpu
from jax.experimental.pallas import tpu_sc as plsc

B = 16384
D = 128
F0 = 25
F1 = 10
NC = 2
NS = 16
NW = NC * NS
NPW = B // NW
LANES = 16
ND = D // LANES

GN = 4
F0P = 26
CH0 = GN * F0P
STRIDE = 128
NCH0 = NPW // GN
SUB1 = 2
CH1 = SUB1 * GN * F1
NCH1 = NPW // (SUB1 * GN)
CHS = 128
NCHS = NPW // CHS

NB = 4
HPW = NPW // 2


def _make_machinery(emb_h, rows_v, out_v, sems):
    def make_task(idx_v, ch):
        def gather(c, b):
            pltpu.async_copy(
                emb_h.at[idx_v.at[pl.ds(c * STRIDE, ch)]],
                rows_v.at[b, pl.ds(0, ch)],
                sems[b],
            )

        def wait_gather(b):
            pltpu.make_async_copy(
                emb_h.at[idx_v.at[pl.ds(0, ch)]],
                rows_v.at[b, pl.ds(0, ch)],
                sems[b],
            ).wait()

        return gather, wait_gather

    def sum_group(b, rbase, fanout, nb):
        def seg(j, i, d):
            return rows_v[b, rbase + j * GN + i, pl.ds(d * LANES, LANES)]

        def jbody(j, accs):
            return tuple(
                accs[i * ND + d] + seg(j, i, d)
                for i in range(GN) for d in range(ND)
            )

        init = tuple(seg(0, i, d) for i in range(GN) for d in range(ND))
        accs = lax.fori_loop(1, fanout, jbody, init, unroll=2)
        for i in range(GN):
            for d in range(ND):
                out_v[nb + i, pl.ds(d * LANES, LANES)] = accs[i * ND + d]

    def run_task(idx_v, nch, ch, reduce_fn, flush_fn=None, npass=1):
        gather, wait_gather = make_task(idx_v, ch)
        for k in range(NB):
            gather(k, k)
        per = nch // npass

        for p in range(npass):
            def quad(q, _):
                for k in range(NB):
                    cl = q * NB + k
                    c = p * per + cl
                    wait_gather(k)
                    reduce_fn(cl, k)

                    @pl.when(c + NB < nch)
                    def _():
                        gather(c + NB, k)

                return 0

            lax.fori_loop(0, per // NB, quad, 0)
            if flush_fn is not None:
                flush_fn(p)

    return sum_group, run_task


def _sc_body_a(nodes_h, n0_h, emb_h, self_h, s0_h,
               idxs_v, idx0_v, rows_v, out_v, sg0, sg1, sg2, sg3):
    wid = lax.axis_index("s") * NC + lax.axis_index("c")
    node_base = wid * NPW

    pltpu.sync_copy(nodes_h.at[pl.ds(wid * NPW, NPW)], idxs_v)
    pltpu.sync_copy(n0_h.at[pl.ds(wid * NCH0 * STRIDE, NCH0 * STRIDE)], idx0_v)

    sum_group, run_task = _make_machinery(emb_h, rows_v, out_v,
                                          (sg0, sg1, sg2, sg3))

    def self_reduce(c, b):
        pltpu.sync_copy(rows_v.at[b, pl.ds(0, CHS)],
                        self_h.at[pl.ds(node_base + c * CHS, CHS)])

    run_task(idxs_v, NCHS, CHS, self_reduce)

    def reduce0(c, b):
        sum_group(b, 0, F0, c * GN)

    def flush0(p):
        pltpu.sync_copy(out_v, s0_h.at[pl.ds(node_base + p * HPW, HPW)])

    run_task(idx0_v, NCH0, CH0, reduce0, flush0, npass=2)


def _sc_body_b(n1_h, emb_h, s1_h, idx1_v, rows_v, out_v, sg0, sg1, sg2, sg3):
    wid = lax.axis_index("s") * NC + lax.axis_index("c")
    node_base = wid * NPW

    pltpu.sync_copy(n1_h.at[pl.ds(wid * NCH1 * STRIDE, NCH1 * STRIDE)], idx1_v)

    sum_group, run_task = _make_machinery(emb_h, rows_v, out_v,
                                          (sg0, sg1, sg2, sg3))

    def reduce1(c, b):
        for s in range(SUB1):
            sum_group(b, s * GN * F1, F1, c * SUB1 * GN + s * GN)

    def flush1(p):
        pltpu.sync_copy(out_v, s1_h.at[pl.ds(node_base, NPW)])

    run_task(idx1_v, NCH1, CH1, reduce1, flush1, npass=1)


def _sc_mesh():
    return plsc.VectorSubcoreMesh(
        core_axis_name="c", subcore_axis_name="s", num_cores=NC, num_subcores=NS
    )


_SEMS = (pltpu.SemaphoreType.DMA,) * NB


@functools.cache
def _sc_gather_a():
    return pl.kernel(
        _sc_body_a,
        out_type=(
            jax.ShapeDtypeStruct((B, D), jnp.float32),
            jax.ShapeDtypeStruct((B, D), jnp.float32),
        ),
        mesh=_sc_mesh(),
        scratch_types=(
            pltpu.VMEM((NPW,), jnp.int32),
            pltpu.VMEM((NCH0 * STRIDE,), jnp.int32),
            pltpu.VMEM((NB, CHS, D), jnp.float32),
            pltpu.VMEM((HPW, D), jnp.float32),
        ) + _SEMS,
    )


@functools.cache
def _sc_gather_b():
    return pl.kernel(
        _sc_body_b,
        out_type=jax.ShapeDtypeStruct((B, D), jnp.float32),
        mesh=_sc_mesh(),
        scratch_types=(
            pltpu.VMEM((NCH1 * STRIDE,), jnp.int32),
            pltpu.VMEM((NB, CH1, D), jnp.float32),
            pltpu.VMEM((NPW, D), jnp.float32),
        ) + _SEMS,
    )


_BLK = 1024


def _layer_body(scale):
    def body(xr, sr, wx, wn, br, o):
        dot = functools.partial(jnp.dot, preferred_element_type=jnp.float32)
        o[...] = jnp.maximum(
            dot(xr[...], wx[...]) + dot(sr[...], wn[...] * scale) + br[...], 0.0
        )

    return body


def _tc_layer(x, s, wx, wn, b, scale):
    big = pl.BlockSpec((_BLK, D), lambda i: (i, 0))
    w = pl.BlockSpec((D, D), lambda i: (0, 0))
    bias = pl.BlockSpec((1, D), lambda i: (0, 0))
    return pl.pallas_call(
        _layer_body(scale),
        grid=(B // _BLK,),
        in_specs=[big, big, w, w, bias],
        out_specs=big,
        out_shape=jax.ShapeDtypeStruct((B, D), jnp.float32),
    )(x, s, wx, wn, b.reshape(1, D))


def kernel(nodes, neigh_samples_0, neigh_samples_1, embedding,
           Ws0, Wn0, b0, Ws1, Wn1, b1):
    nodes1d = nodes.astype(jnp.int32)
    n0 = (neigh_samples_0.astype(jnp.int32)
          .reshape(NW, NCH0, GN, F0).transpose(0, 1, 3, 2))
    n0 = jnp.concatenate([n0, n0[:, :, :F0P - F0, :]], axis=2)
    n0 = n0.reshape(NW, NCH0, CH0)
    n0 = jnp.pad(n0, ((0, 0), (0, 0), (0, STRIDE - CH0))).reshape(-1)
    n1 = (neigh_samples_1.astype(jnp.int32)
          .reshape(NW, NCH1, SUB1, GN, F1).transpose(0, 1, 2, 4, 3))
    n1 = n1.reshape(NW, NCH1, CH1)
    n1 = jnp.pad(n1, ((0, 0), (0, 0), (0, STRIDE - CH1))).reshape(-1)

    self_v, s0 = _sc_gather_a()(nodes1d, n0, embedding)
    s1 = _sc_gather_b()(n1, embedding)
    h = _tc_layer(self_v, s0, Ws0, Wn0, b0, 1.0 / F0)
    return _tc_layer(h, s1, Ws1, Wn1, b1, 1.0 / F1)
